# baseline (speedup 1.0000x reference)
import functools

import jax
import jax.numpy as jnp
from jax import lax
from jax.experimental import pallas as pl
from jax.experimental.pallas import tpu as pltpu
from jax.experimental.pallas import tpu_sc as plsc

N = 10000
E = 320000
D = 128
NLAYER = 3
NOUTER = 3

NC = 2
NS = 16
NW = NC * NS
EPW = E // NW
CH = 40
NCHUNK = EPW // CH
RPT = 640
NPAD = NS * RPT
ZR = 32
NBUF = 6
K = 3
NIDX = 2 * NBUF
R = 6
NOUT = (NCHUNK + NIDX - 1) // NIDX


def _sc_scatter_body(m_hbm, idx_hbm, part_hbm,
                     idxr, rows_v, zbuf_v, agg_sh, gsem, ssem, isem):
    c = lax.axis_index("c")
    s = lax.axis_index("s")
    w = c * NS + s

    for j in range(R):
        pltpu.async_copy(idx_hbm.at[w, j], idxr.at[j], isem.at[j])
    for j in range(K):
        pltpu.make_async_copy(idx_hbm.at[w, j], idxr.at[j], isem.at[j]).wait()
        pltpu.async_copy(m_hbm.at[idxr.at[j, 0]], rows_v.at[j], gsem.at[j])

    def zrow(i, _):
        def zcol(j, _):
            zbuf_v[i, pl.ds(j * 16, 16)] = jnp.zeros((16,), jnp.float32)
            return 0
        return lax.fori_loop(0, D // 16, zcol, 0)
    lax.fori_loop(0, ZR, zrow, 0)

    def zcopy(i, _):
        pltpu.sync_copy(zbuf_v, agg_sh.at[pl.ds(s * RPT + i * ZR, ZR)])
        return 0
    lax.fori_loop(0, RPT // ZR, zcopy, 0)
    plsc.subcore_barrier()

    def outer(t, _):
        for u in range(NIDX):
            g = t * NIDX + u
            b = u % NBUF
            bg = (u + K) % NBUF
            jg = (u + K) % NIDX
            jd = (u + NIDX - (NBUF - K)) % NIDX
            ji = (u + R) % NIDX

            @pl.when(g + K < NCHUNK)
            def _():
                @pl.when(g >= NBUF - K)
                def _():
                    pltpu.make_async_copy(
                        rows_v.at[bg], agg_sh.at[idxr.at[jd, 1]],
                        ssem.at[bg]).wait()
                pltpu.make_async_copy(idx_hbm.at[w, g + K], idxr.at[jg],
                                      isem.at[jg]).wait()
                pltpu.async_copy(m_hbm.at[idxr.at[jg, 0]], rows_v.at[bg],
                                 gsem.at[bg])

            @pl.when(g < NCHUNK)
            def _():
                pltpu.make_async_copy(m_hbm.at[idxr.at[u, 0]],
                                      rows_v.at[b], gsem.at[b]).wait()
                pltpu.async_copy(rows_v.at[b], agg_sh.at[idxr.at[u, 1]],
                                 ssem.at[b], add=True)

            @pl.when(g + R < NCHUNK)
            def _():
                pltpu.async_copy(idx_hbm.at[w, g + R], idxr.at[ji],
                                 isem.at[ji])
        return 0
    lax.fori_loop(0, NOUT, outer, 0)

    for j in range(NCHUNK - NBUF, NCHUNK):
        pltpu.make_async_copy(rows_v.at[j % NBUF],
                              agg_sh.at[idxr.at[j % NIDX, 1]],
                              ssem.at[j % NBUF]).wait()
    plsc.subcore_barrier()

    pltpu.sync_copy(agg_sh.at[pl.ds(s * RPT, RPT)],
                    part_hbm.at[c, pl.ds(s * RPT, RPT)])


_sc_scatter = pl.kernel(
    _sc_scatter_body,
    out_type=jax.ShapeDtypeStruct((NC, NPAD, D), jnp.float32),
    mesh=plsc.VectorSubcoreMesh(core_axis_name="c", subcore_axis_name="s"),
    scratch_types=[
        pltpu.VMEM((NIDX, 2, CH), jnp.int32),
        pltpu.VMEM((NBUF, CH, D), jnp.float32),
        pltpu.VMEM((ZR, D), jnp.float32),
        pltpu.VMEM_SHARED((NPAD, D), jnp.float32),
        pltpu.SemaphoreType.DMA((NBUF,)),
        pltpu.SemaphoreType.DMA((NBUF,)),
        pltpu.SemaphoreType.DMA((NIDX,)),
    ],
)


_BLK = 2000
_GRID = N // _BLK


def _pre_body(x_ref, w_ref, whhT_ref, bhh_ref, m_ref, gh_ref):
    x = x_ref[...]
    m_ref[...] = jnp.dot(x, w_ref[...], preferred_element_type=jnp.float32)
    gh_ref[...] = (jnp.dot(x, whhT_ref[...], preferred_element_type=jnp.float32)
                   + bhh_ref[...])


def _pre_call(x, w, whhT, bhh2):
    return pl.pallas_call(
        _pre_body,
        grid=(_GRID,),
        in_specs=[
            pl.BlockSpec((_BLK, D), lambda i: (i, 0)),
            pl.BlockSpec((D, D), lambda i: (0, 0)),
            pl.BlockSpec((D, 3 * D), lambda i: (0, 0)),
            pl.BlockSpec((1, 3 * D), lambda i: (0, 0)),
        ],
        out_specs=[
            pl.BlockSpec((_BLK, D), lambda i: (i, 0)),
            pl.BlockSpec((_BLK, 3 * D), lambda i: (i, 0)),
        ],
        out_shape=[
            jax.ShapeDtypeStruct((N, D), jnp.float32),
            jax.ShapeDtypeStruct((N, 3 * D), jnp.float32),
        ],
    )(x, w, whhT, bhh2)


def _init_body(ne_ref, wiT_ref, bi_ref, w0_ref, x_ref, m_ref):
    x = (jnp.dot(ne_ref[...], wiT_ref[...], preferred_element_type=jnp.float32)
         + bi_ref[...])
    x_ref[...] = x
    m_ref[...] = jnp.dot(x, w0_ref[...], preferred_element_type=jnp.float32)


def _init_call(node_embed, wiT, bi2, w0):
    return pl.pallas_call(
        _init_body,
        grid=(_GRID,),
        in_specs=[
            pl.BlockSpec((_BLK, D), lambda i: (i, 0)),
            pl.BlockSpec((D, D), lambda i: (0, 0)),
            pl.BlockSpec((1, D), lambda i: (0, 0)),
            pl.BlockSpec((D, D), lambda i: (0, 0)),
        ],
        out_specs=[
            pl.BlockSpec((_BLK, D), lambda i: (i, 0)),
            pl.BlockSpec((_BLK, D), lambda i: (i, 0)),
        ],
        out_shape=[
            jax.ShapeDtypeStruct((N, D), jnp.float32),
            jax.ShapeDtypeStruct((N, D), jnp.float32),
        ],
    )(node_embed, wiT, bi2, w0)


def _gru_gates(part_ref, x, wihT_ref, bih_ref, whhT_ref, bhh_ref):
    agg = part_ref[0] + part_ref[1]
    gi = (jnp.dot(agg, wihT_ref[...], preferred_element_type=jnp.float32)
          + bih_ref[...])
    gh = (jnp.dot(x, whhT_ref[...], preferred_element_type=jnp.float32)
          + bhh_ref[...])
    r = jax.nn.sigmoid(gi[:, :D] + gh[:, :D])
    z = jax.nn.sigmoid(gi[:, D:2 * D] + gh[:, D:2 * D])
    n = jnp.tanh(gi[:, 2 * D:] + r * gh[:, 2 * D:])
    return (1.0 - z) * n + z * x


def _gru_next_body(part_ref, x_ref, wihT_ref, bih_ref, whhT_ref, bhh_ref,
                   wn_ref, x_out_ref, m_out_ref):
    xn = _gru_gates(part_ref, x_ref[...], wihT_ref, bih_ref, whhT_ref, bhh_ref)
    x_out_ref[...] = xn
    m_out_ref[...] = jnp.dot(xn, wn_ref[...], preferred_element_type=jnp.float32)


def _gru_next_call(part, x, wihT, bih2, whhT, bhh2, wn):
    return pl.pallas_call(
        _gru_next_body,
        grid=(_GRID,),
        in_specs=[
            pl.BlockSpec((NC, _BLK, D), lambda i: (0, i, 0)),
            pl.BlockSpec((_BLK, D), lambda i: (i, 0)),
            pl.BlockSpec((D, 3 * D), lambda i: (0, 0)),
            pl.BlockSpec((1, 3 * D), lambda i: (0, 0)),
            pl.BlockSpec((D, 3 * D), lambda i: (0, 0)),
            pl.BlockSpec((1, 3 * D), lambda i: (0, 0)),
            pl.BlockSpec((D, D), lambda i: (0, 0)),
        ],
        out_specs=[
            pl.BlockSpec((_BLK, D), lambda i: (i, 0)),
            pl.BlockSpec((_BLK, D), lambda i: (i, 0)),
        ],
        out_shape=[
            jax.ShapeDtypeStruct((N, D), jnp.float32),
            jax.ShapeDtypeStruct((N, D), jnp.float32),
        ],
    )(part, x, wihT, bih2, whhT, bhh2, wn)


def _gru_final_body(part_ref, x_ref, wihT_ref, bih_ref, whhT_ref, bhh_ref,
                    woT_ref, bo_ref, out_ref, acc_ref):
    k = pl.program_id(0)
    xn = _gru_gates(part_ref, x_ref[...], wihT_ref, bih_ref, whhT_ref, bhh_ref)

    @pl.when(k == 0)
    def _():
        acc_ref[...] = jnp.zeros_like(acc_ref)

    acc_ref[...] += jnp.sum(xn, axis=0, keepdims=True)

    @pl.when(k == pl.num_programs(0) - 1)
    def _():
        out_ref[...] = (jnp.dot(acc_ref[...] / N, woT_ref[...],
                                preferred_element_type=jnp.float32)
                        + bo_ref[...])


def _gru_final_call(part, x, wihT, bih2, whhT, bhh2, woT, bo2):
    return pl.pallas_call(
        _gru_final_body,
        grid=(_GRID,),
        in_specs=[
            pl.BlockSpec((NC, _BLK, D), lambda i: (0, i, 0)),
            pl.BlockSpec((_BLK, D), lambda i: (i, 0)),
            pl.BlockSpec((D, 3 * D), lambda i: (0, 0)),
            pl.BlockSpec((1, 3 * D), lambda i: (0, 0)),
            pl.BlockSpec((D, 3 * D), lambda i: (0, 0)),
            pl.BlockSpec((1, 3 * D), lambda i: (0, 0)),
            pl.BlockSpec((D, D), lambda i: (0, 0)),
            pl.BlockSpec((1, D), lambda i: (0, 0)),
        ],
        out_specs=pl.BlockSpec((1, D), lambda i: (0, 0)),
        out_shape=jax.ShapeDtypeStruct((1, D), jnp.float32),
        scratch_shapes=[pltpu.VMEM((1, D), jnp.float32)],
    )(part, x, wihT, bih2, whhT, bhh2, woT, bo2)


def kernel(node_embed, edge_matrix, Wi, bi, ggnn_w, Wih, Whh, bih, bhh, Wo, bo):
    idx = jnp.stack([edge_matrix[0].reshape(NW, NCHUNK, CH),
                     edge_matrix[1].reshape(NW, NCHUNK, CH)], axis=2)
    whhT = Whh.T
    wihT = Wih.T
    bhh2 = bhh.reshape(1, 3 * D)
    bih2 = bih.reshape(1, 3 * D)
    NIT = NOUTER * NLAYER

    x, m = _init_call(node_embed, Wi.T, bi.reshape(1, D), ggnn_w[0])
    for it in range(NIT):
        part = _sc_scatter(m, idx)
        if it < NIT - 1:
            x, m = _gru_next_call(part, x, wihT, bih2, whhT, bhh2,
                                  ggnn_w[(it + 1) % NLAYER])
        else:
            out = _gru_final_call(part, x, wihT, bih2, whhT, bhh2,
                                  Wo.T, bo.reshape(1, D))
    return out


# speedup vs baseline: 1.0582x; 1.0582x over previous
import functools

import jax
import jax.numpy as jnp
from jax import lax
from jax.experimental import pallas as pl
from jax.experimental.pallas import tpu as pltpu
from jax.experimental.pallas import tpu_sc as plsc

N = 10000
E = 320000
D = 128
NLAYER = 3
NOUTER = 3

NC = 2
NS = 16
NW = NC * NS
EPW = E // NW
CH = 80
NCHUNK = EPW // CH
RPT = 640
NPAD = NS * RPT
ZR = 32
NBUF = 3
K = 2
NIDX = 2 * NBUF
R = 4
NOUT = (NCHUNK + NIDX - 1) // NIDX


def _sc_scatter_body(m_hbm, idx_hbm, part_hbm,
                     idxr, rows_v, zbuf_v, agg_sh, gsem, ssem, isem, zsem):
    c = lax.axis_index("c")
    s = lax.axis_index("s")
    w = c * NS + s

    for j in range(R):
        pltpu.async_copy(idx_hbm.at[w, j], idxr.at[j], isem.at[j])
    for j in range(K):
        pltpu.make_async_copy(idx_hbm.at[w, j], idxr.at[j], isem.at[j]).wait()
        pltpu.async_copy(m_hbm.at[idxr.at[j, 0]], rows_v.at[j], gsem.at[j])

    def zrow(i, _):
        def zcol(j, _):
            zbuf_v[i, pl.ds(j * 16, 16)] = jnp.zeros((16,), jnp.float32)
            return 0
        return lax.fori_loop(0, D // 16, zcol, 0)
    lax.fori_loop(0, ZR, zrow, 0)

    def zcopy(i, _):
        pltpu.async_copy(zbuf_v, agg_sh.at[pl.ds(s * RPT + i * ZR, ZR)], zsem)
        return 0
    lax.fori_loop(0, RPT // ZR, zcopy, 0)

    def zwait(i, _):
        pltpu.make_async_copy(zbuf_v, agg_sh.at[pl.ds(s * RPT + i * ZR, ZR)],
                              zsem).wait()
        return 0
    lax.fori_loop(0, RPT // ZR, zwait, 0)
    plsc.subcore_barrier()

    def outer(t, _):
        for u in range(NIDX):
            g = t * NIDX + u
            b = u % NBUF
            bg = (u + K) % NBUF
            jg = (u + K) % NIDX
            jd = (u + NIDX - (NBUF - K)) % NIDX
            ji = (u + R) % NIDX

            @pl.when(g + K < NCHUNK)
            def _():
                @pl.when(g >= NBUF - K)
                def _():
                    pltpu.make_async_copy(
                        rows_v.at[bg], agg_sh.at[idxr.at[jd, 1]],
                        ssem.at[bg]).wait()
                pltpu.make_async_copy(idx_hbm.at[w, g + K], idxr.at[jg],
                                      isem.at[jg]).wait()
                pltpu.async_copy(m_hbm.at[idxr.at[jg, 0]], rows_v.at[bg],
                                 gsem.at[bg])

            @pl.when(g < NCHUNK)
            def _():
                pltpu.make_async_copy(m_hbm.at[idxr.at[u, 0]],
                                      rows_v.at[b], gsem.at[b]).wait()
                pltpu.async_copy(rows_v.at[b], agg_sh.at[idxr.at[u, 1]],
                                 ssem.at[b], add=True)

            @pl.when(g + R < NCHUNK)
            def _():
                pltpu.async_copy(idx_hbm.at[w, g + R], idxr.at[ji],
                                 isem.at[ji])
        return 0
    lax.fori_loop(0, NOUT, outer, 0)

    for j in range(NCHUNK - NBUF, NCHUNK):
        pltpu.make_async_copy(rows_v.at[j % NBUF],
                              agg_sh.at[idxr.at[j % NIDX, 1]],
                              ssem.at[j % NBUF]).wait()
    plsc.subcore_barrier()

    pltpu.sync_copy(agg_sh.at[pl.ds(s * RPT, RPT)],
                    part_hbm.at[c, pl.ds(s * RPT, RPT)])


_sc_scatter = pl.kernel(
    _sc_scatter_body,
    out_type=jax.ShapeDtypeStruct((NC, NPAD, D), jnp.float32),
    mesh=plsc.VectorSubcoreMesh(core_axis_name="c", subcore_axis_name="s"),
    scratch_types=[
        pltpu.VMEM((NIDX, 2, CH), jnp.int32),
        pltpu.VMEM((NBUF, CH, D), jnp.float32),
        pltpu.VMEM((ZR, D), jnp.float32),
        pltpu.VMEM_SHARED((NPAD, D), jnp.float32),
        pltpu.SemaphoreType.DMA((NBUF,)),
        pltpu.SemaphoreType.DMA((NBUF,)),
        pltpu.SemaphoreType.DMA((NIDX,)),
        pltpu.SemaphoreType.DMA,
    ],
)


_BLK = 2000
_GRID = N // _BLK


def _pre_body(x_ref, w_ref, whhT_ref, bhh_ref, m_ref, gh_ref):
    x = x_ref[...]
    m_ref[...] = jnp.dot(x, w_ref[...], preferred_element_type=jnp.float32)
    gh_ref[...] = (jnp.dot(x, whhT_ref[...], preferred_element_type=jnp.float32)
                   + bhh_ref[...])


def _pre_call(x, w, whhT, bhh2):
    return pl.pallas_call(
        _pre_body,
        grid=(_GRID,),
        in_specs=[
            pl.BlockSpec((_BLK, D), lambda i: (i, 0)),
            pl.BlockSpec((D, D), lambda i: (0, 0)),
            pl.BlockSpec((D, 3 * D), lambda i: (0, 0)),
            pl.BlockSpec((1, 3 * D), lambda i: (0, 0)),
        ],
        out_specs=[
            pl.BlockSpec((_BLK, D), lambda i: (i, 0)),
            pl.BlockSpec((_BLK, 3 * D), lambda i: (i, 0)),
        ],
        out_shape=[
            jax.ShapeDtypeStruct((N, D), jnp.float32),
            jax.ShapeDtypeStruct((N, 3 * D), jnp.float32),
        ],
    )(x, w, whhT, bhh2)


def _init_body(ne_ref, wiT_ref, bi_ref, w0_ref, x_ref, m_ref):
    x = (jnp.dot(ne_ref[...], wiT_ref[...], preferred_element_type=jnp.float32)
         + bi_ref[...])
    x_ref[...] = x
    m_ref[...] = jnp.dot(x, w0_ref[...], preferred_element_type=jnp.float32)


def _init_call(node_embed, wiT, bi2, w0):
    return pl.pallas_call(
        _init_body,
        grid=(_GRID,),
        in_specs=[
            pl.BlockSpec((_BLK, D), lambda i: (i, 0)),
            pl.BlockSpec((D, D), lambda i: (0, 0)),
            pl.BlockSpec((1, D), lambda i: (0, 0)),
            pl.BlockSpec((D, D), lambda i: (0, 0)),
        ],
        out_specs=[
            pl.BlockSpec((_BLK, D), lambda i: (i, 0)),
            pl.BlockSpec((_BLK, D), lambda i: (i, 0)),
        ],
        out_shape=[
            jax.ShapeDtypeStruct((N, D), jnp.float32),
            jax.ShapeDtypeStruct((N, D), jnp.float32),
        ],
    )(node_embed, wiT, bi2, w0)


def _gru_gates(part_ref, x, wihT_ref, bih_ref, whhT_ref, bhh_ref):
    agg = part_ref[0] + part_ref[1]
    gi = (jnp.dot(agg, wihT_ref[...], preferred_element_type=jnp.float32)
          + bih_ref[...])
    gh = (jnp.dot(x, whhT_ref[...], preferred_element_type=jnp.float32)
          + bhh_ref[...])
    r = jax.nn.sigmoid(gi[:, :D] + gh[:, :D])
    z = jax.nn.sigmoid(gi[:, D:2 * D] + gh[:, D:2 * D])
    n = jnp.tanh(gi[:, 2 * D:] + r * gh[:, 2 * D:])
    return (1.0 - z) * n + z * x


def _gru_next_body(part_ref, x_ref, wihT_ref, bih_ref, whhT_ref, bhh_ref,
                   wn_ref, x_out_ref, m_out_ref):
    xn = _gru_gates(part_ref, x_ref[...], wihT_ref, bih_ref, whhT_ref, bhh_ref)
    x_out_ref[...] = xn
    m_out_ref[...] = jnp.dot(xn, wn_ref[...], preferred_element_type=jnp.float32)


def _gru_next_call(part, x, wihT, bih2, whhT, bhh2, wn):
    return pl.pallas_call(
        _gru_next_body,
        grid=(_GRID,),
        in_specs=[
            pl.BlockSpec((NC, _BLK, D), lambda i: (0, i, 0)),
            pl.BlockSpec((_BLK, D), lambda i: (i, 0)),
            pl.BlockSpec((D, 3 * D), lambda i: (0, 0)),
            pl.BlockSpec((1, 3 * D), lambda i: (0, 0)),
            pl.BlockSpec((D, 3 * D), lambda i: (0, 0)),
            pl.BlockSpec((1, 3 * D), lambda i: (0, 0)),
            pl.BlockSpec((D, D), lambda i: (0, 0)),
        ],
        out_specs=[
            pl.BlockSpec((_BLK, D), lambda i: (i, 0)),
            pl.BlockSpec((_BLK, D), lambda i: (i, 0)),
        ],
        out_shape=[
            jax.ShapeDtypeStruct((N, D), jnp.float32),
            jax.ShapeDtypeStruct((N, D), jnp.float32),
        ],
    )(part, x, wihT, bih2, whhT, bhh2, wn)


def _gru_final_body(part_ref, x_ref, wihT_ref, bih_ref, whhT_ref, bhh_ref,
                    woT_ref, bo_ref, out_ref, acc_ref):
    k = pl.program_id(0)
    xn = _gru_gates(part_ref, x_ref[...], wihT_ref, bih_ref, whhT_ref, bhh_ref)

    @pl.when(k == 0)
    def _():
        acc_ref[...] = jnp.zeros_like(acc_ref)

    acc_ref[...] += jnp.sum(xn, axis=0, keepdims=True)

    @pl.when(k == pl.num_programs(0) - 1)
    def _():
        out_ref[...] = (jnp.dot(acc_ref[...] / N, woT_ref[...],
                                preferred_element_type=jnp.float32)
                        + bo_ref[...])


def _gru_final_call(part, x, wihT, bih2, whhT, bhh2, woT, bo2):
    return pl.pallas_call(
        _gru_final_body,
        grid=(_GRID,),
        in_specs=[
            pl.BlockSpec((NC, _BLK, D), lambda i: (0, i, 0)),
            pl.BlockSpec((_BLK, D), lambda i: (i, 0)),
            pl.BlockSpec((D, 3 * D), lambda i: (0, 0)),
            pl.BlockSpec((1, 3 * D), lambda i: (0, 0)),
            pl.BlockSpec((D, 3 * D), lambda i: (0, 0)),
            pl.BlockSpec((1, 3 * D), lambda i: (0, 0)),
            pl.BlockSpec((D, D), lambda i: (0, 0)),
            pl.BlockSpec((1, D), lambda i: (0, 0)),
        ],
        out_specs=pl.BlockSpec((1, D), lambda i: (0, 0)),
        out_shape=jax.ShapeDtypeStruct((1, D), jnp.float32),
        scratch_shapes=[pltpu.VMEM((1, D), jnp.float32)],
    )(part, x, wihT, bih2, whhT, bhh2, woT, bo2)


def kernel(node_embed, edge_matrix, Wi, bi, ggnn_w, Wih, Whh, bih, bhh, Wo, bo):
    idx = jnp.stack([edge_matrix[0].reshape(NW, NCHUNK, CH),
                     edge_matrix[1].reshape(NW, NCHUNK, CH)], axis=2)
    whhT = Whh.T
    wihT = Wih.T
    bhh2 = bhh.reshape(1, 3 * D)
    bih2 = bih.reshape(1, 3 * D)
    NIT = NOUTER * NLAYER

    x, m = _init_call(node_embed, Wi.T, bi.reshape(1, D), ggnn_w[0])
    for it in range(NIT):
        part = _sc_scatter(m, idx)
        if it < NIT - 1:
            x, m = _gru_next_call(part, x, wihT, bih2, whhT, bhh2,
                                  ggnn_w[(it + 1) % NLAYER])
        else:
            out = _gru_final_call(part, x, wihT, bih2, whhT, bhh2,
                                  Wo.T, bo.reshape(1, D))
    return out
